# Initial kernel scaffold; baseline (speedup 1.0000x reference)
import jax
import jax.numpy as jnp
from jax.experimental import pallas as pl


def kernel(elem_queries, attn_mat, keys, adj_values, adj_indices):
    raise NotImplementedError("write your pallas kernel here")


# speedup vs baseline: 28.4662x; 28.4662x over previous
import functools

import jax
import jax.numpy as jnp
import numpy as np
from jax import lax
from jax.experimental import pallas as pl
from jax.experimental.pallas import tpu as pltpu
from jax.experimental.pallas import tpu_sc as plsc

_K = 8
_NRELS = 2
_B = 8
_E = 32
_NNODES = 10000
_NEDGES = 100000
_D = 256
_C = _B * _E

_NW = 32
_EDGES_PAD = 100352
_EPT = _EDGES_PAD // _NW
_CHUNKS = _EPT // 16
_ACC = _NRELS * _B * _E * _E


def _anchor_body(keys_ref, eq_ref, mt_ref, work_ref):
    cc = eq_ref.shape[0]
    scale = 1.0 / float(np.sqrt(_D))
    work_ref[...] = lax.dot_general(
        keys_ref[...], eq_ref[...],
        dimension_numbers=(((1,), (1,)), ((), ())),
        preferred_element_type=jnp.float32,
        precision=lax.Precision.HIGHEST,
    ) * scale

    neg = jnp.float32(-1e30)
    mt_ref[...] = jnp.full((_NNODES, cc), jnp.float32(1e30), jnp.float32)

    vals = []
    for _ in range(_K):
        w = work_ref[...]
        m = jnp.max(w, axis=0, keepdims=True)
        hit = w >= m
        mt_ref[...] = jnp.where(hit, w, mt_ref[...])
        work_ref[...] = jnp.where(hit, neg, w)
        vals.append(m)

    rowmax = vals[0]
    denom = sum(jnp.exp(v - rowmax) for v in vals)
    out = mt_ref[...]
    mt_ref[...] = jnp.where(out < jnp.float32(1e29),
                            jnp.exp(out - rowmax) / denom,
                            jnp.float32(0.0))


def _anchor(keys, eqf):
    cb = _C // 2
    return pl.pallas_call(
        _anchor_body,
        grid=(2,),
        in_specs=[
            pl.BlockSpec((_NNODES, _D), lambda i: (0, 0)),
            pl.BlockSpec((cb, _D), lambda i: (i, 0)),
        ],
        out_specs=pl.BlockSpec((_NNODES, cb), lambda i: (0, i)),
        out_shape=jax.ShapeDtypeStruct((_NNODES, _C), jnp.float32),
        scratch_shapes=[pltpu.VMEM((_NNODES, cb), jnp.float32)],
    )(keys, eqf)


def _bits_body(mt_ref, bits_ref):
    mt = mt_ref[...]
    bits = jnp.zeros((_NNODES, 1), jnp.int32)
    for b in range(_B):
        anyb = jnp.max(mt[:, b * _E:(b + 1) * _E], axis=1, keepdims=True)
        bits = bits + jnp.where(anyb > 0.0, jnp.int32(1 << b), jnp.int32(0))
    bits_ref[...] = bits


def _bits(mt):
    return pl.pallas_call(
        _bits_body,
        out_shape=jax.ShapeDtypeStruct((_NNODES, 1), jnp.int32),
    )(mt)


def _scatter_body(mt_hbm, bits_hbm, rows_hbm, cols_hbm, vals_hbm, rels_hbm,
                  out_hbm,
                  bits_v, rows_v, cols_v, vals_v, rels_v,
                  hr_v, hc_v, hv_v, hm_v, hrel_v,
                  gbuf_v, acc_v, sem):
    wid = lax.axis_index("s") * 2 + lax.axis_index("c")
    base_e = wid * _EPT

    pltpu.sync_copy(bits_hbm, bits_v)
    pltpu.sync_copy(rows_hbm.at[pl.ds(base_e, _EPT)], rows_v)
    pltpu.sync_copy(cols_hbm.at[pl.ds(base_e, _EPT)], cols_v)
    pltpu.sync_copy(vals_hbm.at[pl.ds(base_e, _EPT)], vals_v)
    pltpu.sync_copy(rels_hbm.at[pl.ds(base_e, _EPT)], rels_v)

    def zbody(i, _):
        acc_v[pl.ds(i * 16, 16)] = jnp.zeros((16,), jnp.float32)
        return 0
    lax.fori_loop(0, _ACC // 16, zbody, 0)

    def fbody(ch, off):
        s = ch * 16
        r = rows_v[pl.ds(s, 16)]
        c = cols_v[pl.ds(s, 16)]
        br = plsc.load_gather(bits_v, [r])
        bc = plsc.load_gather(bits_v, [c])
        m = br & bc
        keep = m != 0
        ki = keep.astype(jnp.int32)
        pos = off + plsc.cumsum(ki) - ki
        plsc.store_scatter(hr_v, [pos], r, mask=keep)
        plsc.store_scatter(hc_v, [pos], c, mask=keep)
        plsc.store_scatter(hv_v, [pos], vals_v[pl.ds(s, 16)], mask=keep)
        plsc.store_scatter(hm_v, [pos], m, mask=keep)
        plsc.store_scatter(hrel_v, [pos], rels_v[pl.ds(s, 16)], mask=keep)
        return off + jnp.sum(ki)
    nhits = lax.fori_loop(0, _CHUNKS, fbody, jnp.int32(0))

    lane = lax.iota(jnp.int32, 16)

    def pbody(i, _):
        r = hr_v[pl.ds(i, 16)][0]
        c = hc_v[pl.ds(i, 16)][0]
        v = jnp.clip(hv_v[pl.ds(i, 16)][0], 0.0, 1.0)
        m = hm_v[pl.ds(i, 16)][0]
        rel = hrel_v[pl.ds(i, 16)][0]
        rowidx = jnp.where(lane < 8, r * 8 + lane, c * 8 + (lane - 8))
        pltpu.async_copy(mt_hbm.at[rowidx], gbuf_v, sem).wait()
        for b in range(_B):
            @pl.when(((m >> b) & 1) != 0)
            def _do(b=b):
                tr_lo = gbuf_v[b, pl.ds(0, 16)]
                tr_hi = gbuf_v[b, pl.ds(16, 16)]
                tc_lo = gbuf_v[8 + b, pl.ds(0, 16)]
                tc_hi = gbuf_v[8 + b, pl.ds(16, 16)]
                base = ((rel * _B + b) * _E) * _E
                for ii in range(_E):
                    tri = tr_lo[ii] if ii < 16 else tr_hi[ii - 16]
                    s = v * tri
                    bb = base + ii * _E
                    acc_v[pl.ds(bb, 16)] = acc_v[pl.ds(bb, 16)] + s * tc_lo
                    acc_v[pl.ds(bb + 16, 16)] = (
                        acc_v[pl.ds(bb + 16, 16)] + s * tc_hi)
        return 0
    lax.fori_loop(0, nhits, pbody, 0)

    pltpu.sync_copy(acc_v, out_hbm.at[wid])


@functools.partial(jax.jit, static_argnums=())
def _scatter(mt_flat, bits, rows, cols, vals, rels):
    mesh = plsc.VectorSubcoreMesh(core_axis_name="c", subcore_axis_name="s")
    f = pl.kernel(
        _scatter_body,
        out_type=jax.ShapeDtypeStruct((_NW, _ACC), jnp.float32),
        mesh=mesh,
        compiler_params=pltpu.CompilerParams(
            needs_layout_passes=False, use_tc_tiling_on_sc=False),
        scratch_types=[
            pltpu.VMEM((_NNODES,), jnp.int32),
            pltpu.VMEM((_EPT,), jnp.int32),
            pltpu.VMEM((_EPT,), jnp.int32),
            pltpu.VMEM((_EPT,), jnp.float32),
            pltpu.VMEM((_EPT,), jnp.int32),
            pltpu.VMEM((_EPT + 16,), jnp.int32),
            pltpu.VMEM((_EPT + 16,), jnp.int32),
            pltpu.VMEM((_EPT + 16,), jnp.float32),
            pltpu.VMEM((_EPT + 16,), jnp.int32),
            pltpu.VMEM((_EPT + 16,), jnp.int32),
            pltpu.VMEM((16, 32), jnp.float32),
            pltpu.VMEM((_ACC,), jnp.float32),
            pltpu.SemaphoreType.DMA,
        ],
    )
    return f(mt_flat, bits, rows, cols, vals, rels)


def _loss_body(part_ref, attn_ref, out_ref):
    p = part_ref[...]
    est = jnp.sum(p, axis=0)
    d = attn_ref[...] - est
    s = jnp.sqrt(d)
    l0 = jnp.sum(s[0:_B, :], axis=(0, 1), keepdims=True) / _B
    l1 = jnp.sum(s[_B:2 * _B, :], axis=(0, 1), keepdims=True) / _B
    out_ref[...] = jnp.concatenate([l0, l1], axis=1)


def _loss(partials, attn_t):
    return pl.pallas_call(
        _loss_body,
        out_shape=jax.ShapeDtypeStruct((1, _NRELS), jnp.float32),
    )(partials, attn_t)


def kernel(elem_queries, attn_mat, keys, adj_values, adj_indices):
    eqf = elem_queries.reshape(_C, _D)
    mt = _anchor(keys, eqf)
    bits2 = _bits(mt)
    mt_flat = mt.reshape(_NNODES * _B, _E)
    bits = bits2.reshape(_NNODES)

    pad = _EDGES_PAD - _NEDGES
    rels = jnp.pad(adj_indices[0], (0, pad))
    rows = jnp.pad(adj_indices[1], (0, pad))
    cols = jnp.pad(adj_indices[2], (0, pad))
    vals = jnp.pad(adj_values, (0, pad))

    partials = _scatter(mt_flat, bits, rows, cols, vals, rels)
    attn_t = attn_mat.transpose(1, 0, 2, 3).reshape(_NRELS * _B, _E * _E)
    loss = _loss(partials.reshape(_NW, _NRELS * _B, _E * _E), attn_t)
    return loss.reshape(_NRELS)


# speedup vs baseline: 35.4881x; 1.2467x over previous
import functools

import jax
import jax.numpy as jnp
import numpy as np
from jax import lax
from jax.experimental import pallas as pl
from jax.experimental.pallas import tpu as pltpu
from jax.experimental.pallas import tpu_sc as plsc

_K = 8
_NRELS = 2
_B = 8
_E = 32
_NNODES = 10000
_NEDGES = 100000
_D = 256
_C = _B * _E

_NW = 32
_EPT = 3136
_CHUNKS = _EPT // 16
_ACC = _NRELS * _B * _E * _E


def _anchor_body(keys_ref, eq_ref, mt_ref, bits_ref, work_ref):
    step = pl.program_id(0)
    cc = eq_ref.shape[0]
    scale = 1.0 / float(np.sqrt(_D))
    work_ref[...] = lax.dot_general(
        keys_ref[...], eq_ref[...],
        dimension_numbers=(((1,), (1,)), ((), ())),
        preferred_element_type=jnp.float32,
    ) * scale

    neg = jnp.float32(-1e30)
    mt_ref[...] = jnp.full((_NNODES, cc), jnp.float32(1e30), jnp.float32)

    vals = []
    for _ in range(_K):
        w = work_ref[...]
        m = jnp.max(w, axis=0, keepdims=True)
        hit = w >= m
        mt_ref[...] = jnp.where(hit, w, mt_ref[...])
        work_ref[...] = jnp.where(hit, neg, w)
        vals.append(m)

    rowmax = vals[0]
    denom = sum(jnp.exp(v - rowmax) for v in vals)
    out = mt_ref[...]
    out = jnp.where(out < jnp.float32(1e29),
                    jnp.exp(out - rowmax) / denom,
                    jnp.float32(0.0))
    mt_ref[...] = out

    nb = cc // _E
    sel = (out > 0.0).astype(jnp.float32)
    ciota = lax.broadcasted_iota(jnp.int32, (cc, nb), 0)
    biota = lax.broadcasted_iota(jnp.int32, (cc, nb), 1)
    emat = (ciota // _E == biota).astype(jnp.float32)
    counts = lax.dot_general(
        sel, emat, dimension_numbers=(((1,), (0,)), ((), ())),
        preferred_element_type=jnp.float32)
    shift = lax.broadcasted_iota(jnp.int32, (1, nb), 1) + nb * step
    bitvals = jnp.where(counts > 0.0,
                        jnp.int32(1) << shift, jnp.int32(0))
    contrib = jnp.sum(bitvals, axis=1, keepdims=True)

    @pl.when(step == 0)
    def _init():
        bits_ref[...] = contrib

    @pl.when(step != 0)
    def _acc():
        bits_ref[...] = bits_ref[...] + contrib


def _anchor(keys, eqf):
    cb = _C // 2
    return pl.pallas_call(
        _anchor_body,
        grid=(2,),
        in_specs=[
            pl.BlockSpec((_NNODES, _D), lambda i: (0, 0)),
            pl.BlockSpec((cb, _D), lambda i: (i, 0)),
        ],
        out_specs=(
            pl.BlockSpec((_NNODES, cb), lambda i: (0, i)),
            pl.BlockSpec((_NNODES, 1), lambda i: (0, 0)),
        ),
        out_shape=(
            jax.ShapeDtypeStruct((_NNODES, _C), jnp.float32),
            jax.ShapeDtypeStruct((_NNODES, 1), jnp.int32),
        ),
        scratch_shapes=[pltpu.VMEM((_NNODES, cb), jnp.float32)],
    )(keys, eqf)


def _scatter_body(mt_hbm, bits_hbm, rows_hbm, cols_hbm, vals_hbm, rels_hbm,
                  out_hbm,
                  bits_v, rows_v, cols_v, vals_v, rels_v,
                  hr_v, hc_v, hv_v, hm_v, hrel_v,
                  gbuf_v, acc_v, sem):
    wid = lax.axis_index("s") * 2 + lax.axis_index("c")
    base_e = jnp.minimum(wid * _EPT, _NEDGES - _EPT)
    skip = wid * _EPT - base_e

    pltpu.sync_copy(bits_hbm, bits_v)
    pltpu.sync_copy(rows_hbm.at[pl.ds(base_e, _EPT)], rows_v)
    pltpu.sync_copy(cols_hbm.at[pl.ds(base_e, _EPT)], cols_v)
    pltpu.sync_copy(vals_hbm.at[pl.ds(base_e, _EPT)], vals_v)
    pltpu.sync_copy(rels_hbm.at[pl.ds(base_e, _EPT)], rels_v)

    def zbody(i, _):
        acc_v[pl.ds(i * 16, 16)] = jnp.zeros((16,), jnp.float32)
        return 0
    lax.fori_loop(0, _ACC // 16, zbody, 0)

    lane = lax.iota(jnp.int32, 16)

    def fbody(ch, off):
        s = ch * 16
        r = rows_v[pl.ds(s, 16)]
        c = cols_v[pl.ds(s, 16)]
        br = plsc.load_gather(bits_v, [r])
        bc = plsc.load_gather(bits_v, [c])
        m = br & bc
        keep = (m != 0) & (s + lane >= skip)
        ki = keep.astype(jnp.int32)
        pos = off + plsc.cumsum(ki) - ki
        plsc.store_scatter(hr_v, [pos], r, mask=keep)
        plsc.store_scatter(hc_v, [pos], c, mask=keep)
        plsc.store_scatter(hv_v, [pos], vals_v[pl.ds(s, 16)], mask=keep)
        plsc.store_scatter(hm_v, [pos], m, mask=keep)
        plsc.store_scatter(hrel_v, [pos], rels_v[pl.ds(s, 16)], mask=keep)
        return off + jnp.sum(ki)
    nhits = lax.fori_loop(0, _CHUNKS, fbody, jnp.int32(0))

    def pbody(i, _):
        r = hr_v[pl.ds(i, 16)][0]
        c = hc_v[pl.ds(i, 16)][0]
        v = jnp.clip(hv_v[pl.ds(i, 16)][0], 0.0, 1.0)
        m = hm_v[pl.ds(i, 16)][0]
        rel = hrel_v[pl.ds(i, 16)][0]
        rowidx = jnp.where(lane < 8, r * 8 + lane, c * 8 + (lane - 8))
        pltpu.async_copy(mt_hbm.at[rowidx], gbuf_v, sem).wait()
        for b in range(_B):
            @pl.when(((m >> b) & 1) != 0)
            def _do(b=b):
                tr_lo = gbuf_v[b, pl.ds(0, 16)]
                tr_hi = gbuf_v[b, pl.ds(16, 16)]
                tc_lo = gbuf_v[8 + b, pl.ds(0, 16)]
                tc_hi = gbuf_v[8 + b, pl.ds(16, 16)]
                base = ((rel * _B + b) * _E) * _E
                for ii in range(_E):
                    tri = tr_lo[ii] if ii < 16 else tr_hi[ii - 16]
                    s = v * tri
                    bb = base + ii * _E
                    acc_v[pl.ds(bb, 16)] = acc_v[pl.ds(bb, 16)] + s * tc_lo
                    acc_v[pl.ds(bb + 16, 16)] = (
                        acc_v[pl.ds(bb + 16, 16)] + s * tc_hi)
        return 0
    lax.fori_loop(0, nhits, pbody, 0)

    pltpu.sync_copy(acc_v, out_hbm.at[wid])


def _scatter(mt_flat, bits, rows, cols, vals, rels):
    mesh = plsc.VectorSubcoreMesh(core_axis_name="c", subcore_axis_name="s")
    f = pl.kernel(
        _scatter_body,
        out_type=jax.ShapeDtypeStruct((_NW, _ACC), jnp.float32),
        mesh=mesh,
        compiler_params=pltpu.CompilerParams(
            needs_layout_passes=False, use_tc_tiling_on_sc=False),
        scratch_types=[
            pltpu.VMEM((_NNODES,), jnp.int32),
            pltpu.VMEM((_EPT,), jnp.int32),
            pltpu.VMEM((_EPT,), jnp.int32),
            pltpu.VMEM((_EPT,), jnp.float32),
            pltpu.VMEM((_EPT,), jnp.int32),
            pltpu.VMEM((_EPT + 16,), jnp.int32),
            pltpu.VMEM((_EPT + 16,), jnp.int32),
            pltpu.VMEM((_EPT + 16,), jnp.float32),
            pltpu.VMEM((_EPT + 16,), jnp.int32),
            pltpu.VMEM((_EPT + 16,), jnp.int32),
            pltpu.VMEM((16, 32), jnp.float32),
            pltpu.VMEM((_ACC,), jnp.float32),
            pltpu.SemaphoreType.DMA,
        ],
    )
    return f(mt_flat, bits, rows, cols, vals, rels)


def _loss_body(part_ref, attn_ref, out_ref):
    p = part_ref[...]
    est = jnp.sum(p, axis=0)
    d = attn_ref[...] - est
    s = jnp.sqrt(d)
    l0 = jnp.sum(s[0:_B, :], axis=(0, 1), keepdims=True) / _B
    l1 = jnp.sum(s[_B:2 * _B, :], axis=(0, 1), keepdims=True) / _B
    out_ref[...] = jnp.concatenate([l0, l1], axis=1)


def _loss(partials, attn_t):
    return pl.pallas_call(
        _loss_body,
        out_shape=jax.ShapeDtypeStruct((1, _NRELS), jnp.float32),
    )(partials, attn_t)


def kernel(elem_queries, attn_mat, keys, adj_values, adj_indices):
    eqf = elem_queries.reshape(_C, _D)
    mt, bits2 = _anchor(keys, eqf)
    mt_flat = mt.reshape(_NNODES * _B, _E)
    bits = bits2.reshape(_NNODES)

    partials = _scatter(mt_flat, bits, adj_indices[1], adj_indices[2],
                        adj_values, adj_indices[0])
    attn_t = attn_mat.transpose(1, 0, 2, 3).reshape(_NRELS * _B, _E * _E)
    loss = _loss(partials.reshape(_NW, _NRELS * _B, _E * _E), attn_t)
    return loss.reshape(_NRELS)


# speedup vs baseline: 40.9481x; 1.1539x over previous
import functools

import jax
import jax.numpy as jnp
import numpy as np
from jax import lax
from jax.experimental import pallas as pl
from jax.experimental.pallas import tpu as pltpu
from jax.experimental.pallas import tpu_sc as plsc

_K = 8
_NRELS = 2
_B = 8
_E = 32
_NNODES = 10000
_NEDGES = 100000
_D = 256
_C = _B * _E

_NW = 32
_EPT = 3136
_CHUNKS = _EPT // 16
_ACC = _NRELS * _B * _E * _E


def _anchor_body(keys_ref, eq_ref, mt_ref, bits_ref, work_ref):
    step = pl.program_id(0)
    cc = eq_ref.shape[0]
    scale = 1.0 / float(np.sqrt(_D))
    work_ref[...] = lax.dot_general(
        keys_ref[...], eq_ref[...],
        dimension_numbers=(((1,), (1,)), ((), ())),
        preferred_element_type=jnp.float32,
    ) * scale

    neg = jnp.float32(-1e30)
    mt_ref[...] = jnp.full((_NNODES, cc), jnp.float32(1e30), jnp.float32)

    vals = []
    for _ in range(_K):
        w = work_ref[...]
        m = jnp.max(w, axis=0, keepdims=True)
        hit = w >= m
        mt_ref[...] = jnp.where(hit, w, mt_ref[...])
        work_ref[...] = jnp.where(hit, neg, w)
        vals.append(m)

    rowmax = vals[0]
    denom = sum(jnp.exp(v - rowmax) for v in vals)
    out = mt_ref[...]
    out = jnp.where(out < jnp.float32(1e29),
                    jnp.exp(out - rowmax) / denom,
                    jnp.float32(0.0))
    mt_ref[...] = out

    nb = cc // _E
    sel = (out > 0.0).astype(jnp.float32)
    ciota = lax.broadcasted_iota(jnp.int32, (cc, nb), 0)
    biota = lax.broadcasted_iota(jnp.int32, (cc, nb), 1)
    emat = (ciota // _E == biota).astype(jnp.float32)
    counts = lax.dot_general(
        sel, emat, dimension_numbers=(((1,), (0,)), ((), ())),
        preferred_element_type=jnp.float32)
    shift = lax.broadcasted_iota(jnp.int32, (1, nb), 1) + nb * step
    bitvals = jnp.where(counts > 0.0,
                        jnp.int32(1) << shift, jnp.int32(0))
    contrib = jnp.sum(bitvals, axis=1, keepdims=True)

    @pl.when(step == 0)
    def _init():
        bits_ref[...] = contrib

    @pl.when(step != 0)
    def _acc():
        bits_ref[...] = bits_ref[...] + contrib


def _anchor(keys, eqf):
    cb = _C // 2
    return pl.pallas_call(
        _anchor_body,
        grid=(2,),
        in_specs=[
            pl.BlockSpec((_NNODES, _D), lambda i: (0, 0)),
            pl.BlockSpec((cb, _D), lambda i: (i, 0)),
        ],
        out_specs=(
            pl.BlockSpec((_NNODES, cb), lambda i: (0, i)),
            pl.BlockSpec((_NNODES, 1), lambda i: (0, 0)),
        ),
        out_shape=(
            jax.ShapeDtypeStruct((_NNODES, _C), jnp.float32),
            jax.ShapeDtypeStruct((_NNODES, 1), jnp.int32),
        ),
        scratch_shapes=[pltpu.VMEM((_NNODES, cb), jnp.float32)],
    )(keys, eqf)


_NBUF = 32


def _scatter_body(mt_hbm, bits_hbm, adj_hbm, vals_hbm,
                  out_hbm,
                  bits_v, rows_v, cols_v, vals_v, rels_v,
                  hr_v, hc_v, hv_v, hm_v, hrel_v,
                  gbuf_v, acc_v, sem, gsem):
    wid = lax.axis_index("s") * 2 + lax.axis_index("c")
    base_e = jnp.minimum(wid * _EPT, _NEDGES - _EPT)
    skip = wid * _EPT - base_e

    pltpu.async_copy(bits_hbm, bits_v, sem)
    pltpu.async_copy(adj_hbm.at[1, pl.ds(base_e, _EPT)], rows_v, sem)
    pltpu.async_copy(adj_hbm.at[2, pl.ds(base_e, _EPT)], cols_v, sem)
    pltpu.async_copy(adj_hbm.at[0, pl.ds(base_e, _EPT)], rels_v, sem)
    pltpu.async_copy(vals_hbm.at[pl.ds(base_e, _EPT)], vals_v, sem)

    def zbody(i, _):
        acc_v[pl.ds(i * 16, 16)] = jnp.zeros((16,), jnp.float32)
        return 0
    lax.fori_loop(0, _ACC // 16, zbody, 0)

    pltpu.make_async_copy(bits_hbm, bits_v, sem).wait()
    pltpu.make_async_copy(adj_hbm.at[1, pl.ds(base_e, _EPT)], rows_v,
                          sem).wait()
    pltpu.make_async_copy(adj_hbm.at[2, pl.ds(base_e, _EPT)], cols_v,
                          sem).wait()
    pltpu.make_async_copy(adj_hbm.at[0, pl.ds(base_e, _EPT)], rels_v,
                          sem).wait()
    pltpu.make_async_copy(vals_hbm.at[pl.ds(base_e, _EPT)], vals_v,
                          sem).wait()

    lane = lax.iota(jnp.int32, 16)

    def fbody(ch, off):
        s = ch * 16
        r = rows_v[pl.ds(s, 16)]
        c = cols_v[pl.ds(s, 16)]
        br = plsc.load_gather(bits_v, [r])
        bc = plsc.load_gather(bits_v, [c])
        m = br & bc
        keep = (m != 0) & (s + lane >= skip)
        ki = keep.astype(jnp.int32)
        pos = off + plsc.cumsum(ki) - ki
        plsc.store_scatter(hr_v, [pos], r, mask=keep)
        plsc.store_scatter(hc_v, [pos], c, mask=keep)
        plsc.store_scatter(hv_v, [pos], vals_v[pl.ds(s, 16)], mask=keep)
        plsc.store_scatter(hm_v, [pos], m, mask=keep)
        plsc.store_scatter(hrel_v, [pos], rels_v[pl.ds(s, 16)], mask=keep)
        return off + jnp.sum(ki)
    nhits = lax.fori_loop(0, _CHUNKS, fbody, jnp.int32(0))

    def wave(carry):
        start = carry
        cnt = jnp.minimum(nhits - start, _NBUF)

        def issue(j, _):
            i = start + j
            r = hr_v[pl.ds(i, 16)][0]
            c = hc_v[pl.ds(i, 16)][0]
            rowidx = jnp.where(lane < 8, r * 8 + lane, c * 8 + (lane - 8))
            pltpu.async_copy(mt_hbm.at[rowidx], gbuf_v.at[j], gsem)
            return 0
        lax.fori_loop(0, cnt, issue, 0)

        def drain(j, _):
            pltpu.make_async_copy(mt_hbm.at[pl.ds(0, 16)], gbuf_v.at[0],
                                  gsem).wait()
            return 0
        lax.fori_loop(0, cnt, drain, 0)

        def proc(j, _):
            i = start + j
            v = jnp.clip(hv_v[pl.ds(i, 16)][0], 0.0, 1.0)
            m = hm_v[pl.ds(i, 16)][0]
            rel = hrel_v[pl.ds(i, 16)][0]
            for b in range(_B):
                @pl.when(((m >> b) & 1) != 0)
                def _do(b=b):
                    tr_lo = gbuf_v[j, b, pl.ds(0, 16)]
                    tr_hi = gbuf_v[j, b, pl.ds(16, 16)]
                    tc_lo = gbuf_v[j, 8 + b, pl.ds(0, 16)]
                    tc_hi = gbuf_v[j, 8 + b, pl.ds(16, 16)]
                    base = ((b * _NRELS + rel) * _E) * _E
                    for ii in range(_E):
                        tri = tr_lo[ii] if ii < 16 else tr_hi[ii - 16]
                        s = v * tri
                        bb = base + ii * _E
                        acc_v[pl.ds(bb, 16)] = (
                            acc_v[pl.ds(bb, 16)] + s * tc_lo)
                        acc_v[pl.ds(bb + 16, 16)] = (
                            acc_v[pl.ds(bb + 16, 16)] + s * tc_hi)
            return 0
        lax.fori_loop(0, cnt, proc, 0)
        return start + cnt

    lax.while_loop(lambda s: s < nhits, wave, jnp.int32(0))

    pltpu.sync_copy(acc_v, out_hbm.at[wid])


def _scatter(mt_flat, bits, adj_indices, vals):
    mesh = plsc.VectorSubcoreMesh(core_axis_name="c", subcore_axis_name="s")
    f = pl.kernel(
        _scatter_body,
        out_type=jax.ShapeDtypeStruct((_NW, _ACC), jnp.float32),
        mesh=mesh,
        compiler_params=pltpu.CompilerParams(
            needs_layout_passes=False, use_tc_tiling_on_sc=False),
        scratch_types=[
            pltpu.VMEM((_NNODES,), jnp.int32),
            pltpu.VMEM((_EPT,), jnp.int32),
            pltpu.VMEM((_EPT,), jnp.int32),
            pltpu.VMEM((_EPT,), jnp.float32),
            pltpu.VMEM((_EPT,), jnp.int32),
            pltpu.VMEM((_EPT + 16,), jnp.int32),
            pltpu.VMEM((_EPT + 16,), jnp.int32),
            pltpu.VMEM((_EPT + 16,), jnp.float32),
            pltpu.VMEM((_EPT + 16,), jnp.int32),
            pltpu.VMEM((_EPT + 16,), jnp.int32),
            pltpu.VMEM((_NBUF, 16, 32), jnp.float32),
            pltpu.VMEM((_ACC,), jnp.float32),
            pltpu.SemaphoreType.DMA,
            pltpu.SemaphoreType.DMA,
        ],
    )
    return f(mt_flat, bits, adj_indices, vals)


def _loss_body(part_ref, attn_ref, out_ref):
    p = part_ref[...]
    est = jnp.sum(p, axis=0)
    est3 = est.reshape(_B, _NRELS, _E * _E)
    attn3 = attn_ref[...].reshape(_B, _NRELS, _E * _E)
    s = jnp.sqrt(attn3 - est3)
    l0 = jnp.sum(s[:, 0, :], axis=(0, 1), keepdims=True) / _B
    l1 = jnp.sum(s[:, 1, :], axis=(0, 1), keepdims=True) / _B
    out_ref[...] = jnp.concatenate([l0, l1], axis=1)


def _loss(partials, attn_mat):
    return pl.pallas_call(
        _loss_body,
        out_shape=jax.ShapeDtypeStruct((1, _NRELS), jnp.float32),
    )(partials, attn_mat)


def kernel(elem_queries, attn_mat, keys, adj_values, adj_indices):
    eqf = elem_queries.reshape(_C, _D)
    mt, bits2 = _anchor(keys, eqf)
    mt_flat = mt.reshape(_NNODES * _B, _E)
    bits = bits2.reshape(_NNODES)

    partials = _scatter(mt_flat, bits, adj_indices, adj_values)
    loss = _loss(partials.reshape(_NW, _B * _NRELS, _E * _E), attn_mat)
    return loss.reshape(_NRELS)
